# Initial kernel scaffold; baseline (speedup 1.0000x reference)
import jax
import jax.numpy as jnp
from jax.experimental import pallas as pl


def kernel(inputs, targets):
    raise NotImplementedError("write your pallas kernel here")


# speedup vs baseline: 1.0107x; 1.0107x over previous
import functools

import jax
import jax.numpy as jnp
from jax import lax
from jax.experimental import pallas as pl
from jax.experimental.pallas import tpu as pltpu
from jax.experimental.pallas import tpu_sc as plsc

B = 1024
V = 100000

_info = plsc.get_sparse_core_info()
_NC, _NS, _L = _info.num_cores, _info.num_subcores, _info.num_lanes
_NW = _NC * _NS
_BPW = B // _NW

_mesh = plsc.VectorSubcoreMesh(core_axis_name="c", subcore_axis_name="s")


@functools.partial(
    pl.kernel,
    mesh=_mesh,
    out_type=jax.ShapeDtypeStruct((B,), jnp.float32),
    scratch_types=[
        pltpu.VMEM((_BPW,), jnp.int32),
        pltpu.VMEM((_BPW,), jnp.int32),
        pltpu.VMEM((_BPW,), jnp.float32),
        pltpu.SemaphoreType.DMA,
    ],
)
def _gather_loss(flat_hbm, tgt_hbm, out_hbm, tgt_v, idx_v, val_v, sem):
    wid = lax.axis_index("s") * _NC + lax.axis_index("c")
    base = wid * _BPW
    pltpu.sync_copy(tgt_hbm.at[pl.ds(base, _BPW)], tgt_v)
    for j in range(_BPW // _L):
        t = tgt_v[pl.ds(j * _L, _L)]
        rows = (base + j * _L) + lax.broadcasted_iota(jnp.int32, (_L,), 0)
        idx_v[pl.ds(j * _L, _L)] = rows * V + t
    pltpu.async_copy(flat_hbm.at[idx_v], val_v, sem).wait()
    pltpu.sync_copy(val_v, out_hbm.at[pl.ds(base, _BPW)])


def kernel(inputs, targets):
    flat = inputs.reshape(-1)
    tgt = targets.astype(jnp.int32)
    return _gather_loss(flat, tgt)


# speedup vs baseline: 2.3867x; 2.3615x over previous
import functools

import jax
import jax.numpy as jnp
from jax import lax
from jax.experimental import pallas as pl
from jax.experimental.pallas import tpu as pltpu
from jax.experimental.pallas import tpu_sc as plsc

B = 1024
V = 100000

_info = plsc.get_sparse_core_info()
_NC, _NS, _L = _info.num_cores, _info.num_subcores, _info.num_lanes
_NW = _NC * _NS
_BPW = B // _NW

_mesh = plsc.VectorSubcoreMesh(core_axis_name="c", subcore_axis_name="s")


@functools.partial(
    pl.kernel,
    mesh=_mesh,
    out_type=jax.ShapeDtypeStruct((B,), jnp.float32),
    scratch_types=[
        pltpu.VMEM((_BPW,), jnp.int32),
        pltpu.VMEM((_BPW, 128), jnp.float32),
        pltpu.VMEM((_BPW,), jnp.float32),
        pltpu.SemaphoreType.DMA,
    ],
    compiler_params=pltpu.CompilerParams(
        use_tc_tiling_on_sc=True, needs_layout_passes=False
    ),
)
def _gather_loss(in_hbm, tgt_hbm, out_hbm, tgt_v, rows_v, val_v, sem):
    wid = lax.axis_index("s") * _NC + lax.axis_index("c")
    base = wid * _BPW
    pltpu.sync_copy(tgt_hbm.at[pl.ds(base, _BPW)], tgt_v)
    iota = lax.broadcasted_iota(jnp.int32, (_L,), 0)
    copies = []
    for k in range(_BPW):
        chunk = tgt_v[pl.ds((k // _L) * _L, _L)]
        t = jnp.sum(jnp.where(iota == (k % _L), chunk, 0))
        c0 = pl.multiple_of((t // 128) * 128, 128)
        cp = pltpu.make_async_copy(
            in_hbm.at[pl.ds(base + k, 1), pl.ds(c0, 128)],
            rows_v.at[pl.ds(k, 1), :],
            sem,
        )
        cp.start()
        copies.append(cp)
    for cp in copies:
        cp.wait()
    for j in range(_BPW // _L):
        ridx = lax.broadcasted_iota(jnp.int32, (_L,), 0) + j * _L
        cidx = tgt_v[pl.ds(j * _L, _L)] % 128
        val_v[pl.ds(j * _L, _L)] = plsc.load_gather(rows_v, [ridx, cidx])
    pltpu.sync_copy(val_v, out_hbm.at[pl.ds(base, _BPW)])


def kernel(inputs, targets):
    tgt = targets.astype(jnp.int32)
    return _gather_loss(inputs, tgt)


# speedup vs baseline: 2.3906x; 1.0017x over previous
import functools

import jax
import jax.numpy as jnp
from jax import lax
from jax.experimental import pallas as pl
from jax.experimental.pallas import tpu as pltpu
from jax.experimental.pallas import tpu_sc as plsc

B = 1024
V = 100000

_info = plsc.get_sparse_core_info()
_NC, _NS, _L = _info.num_cores, _info.num_subcores, _info.num_lanes
_NW = _NC * _NS
_BPW = B // _NW

_mesh = plsc.VectorSubcoreMesh(core_axis_name="c", subcore_axis_name="s")


@functools.partial(
    pl.kernel,
    mesh=_mesh,
    out_type=jax.ShapeDtypeStruct((B,), jnp.float32),
    scratch_types=[
        pltpu.VMEM((_BPW,), jnp.int32),
        pltpu.VMEM((_BPW, 128), jnp.float32),
        pltpu.VMEM((_BPW,), jnp.float32),
        pltpu.SemaphoreType.DMA,
    ],
    compiler_params=pltpu.CompilerParams(
        use_tc_tiling_on_sc=True,
        needs_layout_passes=False,
        skip_device_barrier=True,
    ),
)
def _gather_loss(in_hbm, tgt_hbm, out_hbm, tgt_v, rows_v, val_v, sem):
    wid = lax.axis_index("s") * _NC + lax.axis_index("c")
    base = wid * _BPW
    pltpu.sync_copy(tgt_hbm.at[pl.ds(base, _BPW)], tgt_v)
    iota = lax.broadcasted_iota(jnp.int32, (_L,), 0)
    copies = []
    for k in range(_BPW):
        chunk = tgt_v[pl.ds((k // _L) * _L, _L)]
        t = jnp.sum(jnp.where(iota == (k % _L), chunk, 0))
        c0 = pl.multiple_of((t // 128) * 128, 128)
        cp = pltpu.make_async_copy(
            in_hbm.at[pl.ds(base + k, 1), pl.ds(c0, 128)],
            rows_v.at[pl.ds(k, 1), :],
            sem,
        )
        cp.start()
        copies.append(cp)
    for cp in copies:
        cp.wait()
    for j in range(_BPW // _L):
        ridx = lax.broadcasted_iota(jnp.int32, (_L,), 0) + j * _L
        cidx = tgt_v[pl.ds(j * _L, _L)] % 128
        val_v[pl.ds(j * _L, _L)] = plsc.load_gather(rows_v, [ridx, cidx])
    pltpu.sync_copy(val_v, out_hbm.at[pl.ds(base, _BPW)])


def kernel(inputs, targets):
    tgt = targets.astype(jnp.int32)
    return _gather_loss(inputs, tgt)
